# baseline (speedup 1.0000x reference)
import functools

import numpy as np
import jax
import jax.numpy as jnp
from jax import lax
from jax.experimental import pallas as pl
from jax.experimental.pallas import tpu as pltpu
from jax.experimental.pallas import tpu_sc as plsc

_NUM_ENTITIES = 1000000
_NUM_RELATIONS = 1000
_D = 64
_BATCH = 16384
_NEG = 2

_TILES = 32
_NC = 2
_ROWS = _BATCH // _TILES
_CHUNK = 128
_NCHUNKS = _ROWS // _CHUNK
_STREAMS = 7


def _neg_index_planes():
    nk1, nk2 = jax.random.split(jax.random.key(42), 2)
    fo1 = jax.random.randint(nk1, (_NEG * _BATCH,), 0, _NUM_ENTITIES, dtype=jnp.int32)
    fs2 = jax.random.randint(nk2, (_NEG * _BATCH,), 0, _NUM_ENTITIES, dtype=jnp.int32)
    return fo1[0::2], fo1[1::2], fs2[0::2], fs2[1::2]


def _sc_body(idx_hbm, etab, rtab, ts_hbm, ps_hbm,
             idx_v, b_s, b_p, b_o, b1e, b1o, b2e, b2o, ts_v, ps_v, sem0, sem1):
    wid = lax.axis_index("s") * _NC + lax.axis_index("c")

    pltpu.sync_copy(idx_hbm.at[wid], idx_v)

    bufs = (b_s, b_p, b_o, b1e, b1o, b2e, b2o)
    tabs = (etab, rtab, etab, etab, etab, etab, etab)
    sems = (sem0, sem1)

    def fire(c):
        slot = c % 2
        return [
            pltpu.async_copy(
                tabs[j].at[idx_v.at[j * _NCHUNKS + c]],
                bufs[j].at[pl.ds(slot * _CHUNK, _CHUNK)],
                sems[slot])
            for j in range(_STREAMS)
        ]

    iota = lax.iota(jnp.int32, 16)
    zero = jnp.zeros((16,), jnp.float32)
    sums = (zero, zero, zero)

    handles = {0: fire(0)}
    for c in range(_NCHUNKS):
        if c + 1 < _NCHUNKS:
            handles[c + 1] = fire(c + 1)
        for h in handles.pop(c):
            h.wait()
        slot = c % 2

        def blk_body(b, carry, _c=c, _slot=slot):
            st, s1, s2 = carry
            rows = _slot * _CHUNK + b * 16 + iota

            def d_body(j, accs):
                at, a1e, a1o, a2e, a2o = accs
                for t in range(4):
                    d = j * 4 + t
                    col = jnp.full((16,), 0, jnp.int32) + d
                    sv = plsc.load_gather(b_s, [rows, col])
                    pv = plsc.load_gather(b_p, [rows, col])
                    ov = plsc.load_gather(b_o, [rows, col])
                    sp = sv * pv
                    po = pv * ov
                    at = at + sp * ov
                    a1e = a1e + sp * plsc.load_gather(b1e, [rows, col])
                    a1o = a1o + sp * plsc.load_gather(b1o, [rows, col])
                    a2e = a2e + po * plsc.load_gather(b2e, [rows, col])
                    a2o = a2o + po * plsc.load_gather(b2o, [rows, col])
                return (at, a1e, a1o, a2e, a2o)

            at, a1e, a1o, a2e, a2o = lax.fori_loop(
                0, 16, d_body, (zero, zero, zero, zero, zero))

            ts_v[pl.ds(_c * _CHUNK + b * 16, 16)] = at
            st = st + 1.0 / (1.0 + jnp.exp(-at))
            s1 = s1 + 1.0 / (1.0 + jnp.exp(a1e)) + 1.0 / (1.0 + jnp.exp(a1o))
            s2 = s2 + 1.0 / (1.0 + jnp.exp(a2e)) + 1.0 / (1.0 + jnp.exp(a2o))
            return (st, s1, s2)

        sums = lax.fori_loop(0, _CHUNK // 16, blk_body, sums)

    ps_v[0, :] = sums[0]
    ps_v[1, :] = sums[1]
    ps_v[2, :] = sums[2]
    pltpu.sync_copy(ts_v, ts_hbm.at[pl.ds(wid * _ROWS, _ROWS)])
    pltpu.sync_copy(ps_v, ps_hbm.at[wid])


_FUSED = pl.kernel(
    _sc_body,
    out_type=(
        jax.ShapeDtypeStruct((_BATCH,), jnp.float32),
        jax.ShapeDtypeStruct((_TILES, 3, 16), jnp.float32),
    ),
    mesh=plsc.VectorSubcoreMesh(core_axis_name="c", subcore_axis_name="s"),
    compiler_params=pltpu.CompilerParams(
        needs_layout_passes=False, use_tc_tiling_on_sc=False),
    scratch_types=[
        pltpu.VMEM((_STREAMS * _NCHUNKS, _CHUNK), jnp.int32),
        pltpu.VMEM((2 * _CHUNK, _D), jnp.float32),
        pltpu.VMEM((2 * _CHUNK, _D), jnp.float32),
        pltpu.VMEM((2 * _CHUNK, _D), jnp.float32),
        pltpu.VMEM((2 * _CHUNK, _D), jnp.float32),
        pltpu.VMEM((2 * _CHUNK, _D), jnp.float32),
        pltpu.VMEM((2 * _CHUNK, _D), jnp.float32),
        pltpu.VMEM((2 * _CHUNK, _D), jnp.float32),
        pltpu.VMEM((_ROWS,), jnp.float32),
        pltpu.VMEM((3, 16), jnp.float32),
        pltpu.SemaphoreType.DMA,
        pltpu.SemaphoreType.DMA,
    ],
)


def kernel(inputs, entity_table, relation_table):
    f1e, f1o, f2e, f2o = _neg_index_planes()
    idx_all = jnp.stack(
        [inputs[:, 0], inputs[:, 1], inputs[:, 2], f1e, f1o, f2e, f2o],
        axis=0)
    idx_all = idx_all.reshape(_STREAMS, _TILES, _NCHUNKS * _CHUNK)
    idx_all = idx_all.transpose(1, 0, 2).reshape(
        _TILES, _STREAMS * _NCHUNKS, _CHUNK)

    true_score, partials = _FUSED(idx_all, entity_table, relation_table)

    ssum = jnp.sum(partials, axis=(0, 2))
    mean_sig_true = ssum[0] * (1.0 / _BATCH)
    m1 = ssum[1] * (1.0 / (_NEG * _BATCH))
    m2 = ssum[2] * (1.0 / (_NEG * _BATCH))
    loss = 1.0 - mean_sig_true / 2.0 - (m1 + m2) / 4.0
    return (true_score, loss)
